# baseline (speedup 1.0000x reference)
import functools

import jax
import jax.numpy as jnp
from jax import lax
from jax.experimental import pallas as pl
from jax.experimental.pallas import tpu as pltpu
from jax.experimental.pallas import tpu_sc as plsc

N_NODES = 10000
D = 128
N_REL = 8
E = 320000
R_CAT = N_REL + 1

NC, NS = 2, 16
NW = NC * NS
EPW = E // NW
CHUNK = 125
NCH = EPW // CHUNK
NBUF = 2
NGRP = NCH // NBUF
ROWS_PT = 624
ROWS_TAIL = N_NODES - NS * ROWS_PT

_MESH = plsc.VectorSubcoreMesh(core_axis_name="c", subcore_axis_name="s")


def _mm_body(x_ref, w_ref, o_ref, xr_ref):
    h = jnp.dot(x_ref[...], w_ref[...], preferred_element_type=jnp.float32)
    o_ref[...] = h
    xr_ref[...] = h[:, (R_CAT - 1) * D:]


def _tc_matmul(x, wcat2d, nb=10):
    blk = N_NODES // nb
    return pl.pallas_call(
        _mm_body,
        grid=(nb,),
        in_specs=[
            pl.BlockSpec((blk, D), lambda b: (b, 0)),
            pl.BlockSpec((D, R_CAT * D), lambda b: (0, 0)),
        ],
        out_specs=[
            pl.BlockSpec((blk, R_CAT * D), lambda b: (b, 0)),
            pl.BlockSpec((blk, D), lambda b: (b, 0)),
        ],
        out_shape=[
            jax.ShapeDtypeStruct((N_NODES, R_CAT * D), jnp.float32),
            jax.ShapeDtypeStruct((N_NODES, D), jnp.float32),
        ],
    )(x, wcat2d)


def _sc_body(hall, gidx_hbm, dst_hbm, zeros_hbm, out_hbm,
             ig_v, id_v, rows_v, agg_sp, igsems, idsems, gsems, ssems):
    c = lax.axis_index("c")
    s = lax.axis_index("s")
    wid = s * NC + c

    pltpu.sync_copy(zeros_hbm.at[pl.ds(s * ROWS_PT, ROWS_PT)],
                    agg_sp.at[pl.ds(s * ROWS_PT, ROWS_PT)])

    @pl.when(s == 0)
    def _init_tail():
        pltpu.sync_copy(zeros_hbm.at[pl.ds(NS * ROWS_PT, ROWS_TAIL)],
                        agg_sp.at[pl.ds(NS * ROWS_PT, ROWS_TAIL)])

    plsc.subcore_barrier()

    def _idx_start(g, slot):
        pltpu.async_copy(gidx_hbm.at[wid, g], ig_v.at[slot], igsems[slot])
        pltpu.async_copy(dst_hbm.at[wid, g], id_v.at[slot], idsems[slot])

    def _idx_wait(g, slot):
        pltpu.make_async_copy(gidx_hbm.at[wid, g], ig_v.at[slot],
                              igsems[slot]).wait()
        pltpu.make_async_copy(dst_hbm.at[wid, g], id_v.at[slot],
                              idsems[slot]).wait()

    def _start_gather(slot, b):
        pltpu.async_copy(hall.at[ig_v.at[slot, b]], rows_v.at[b], gsems[b])

    def _wait_gather(slot, b):
        pltpu.make_async_copy(hall.at[ig_v.at[slot, b]], rows_v.at[b],
                              gsems[b]).wait()

    def _start_scatter(slot, b):
        pltpu.async_copy(rows_v.at[b], agg_sp.at[id_v.at[slot, b]],
                         ssems[b], add=True)

    def _wait_scatter(slot, b):
        pltpu.make_async_copy(rows_v.at[b], agg_sp.at[id_v.at[slot, b]],
                              ssems[b]).wait()

    _idx_start(0, 0)
    _idx_start(1, 1)
    _idx_wait(0, 0)
    for b in range(NBUF):
        _start_gather(0, b)

    def pair(i, carry):
        for slot in range(2):
            g = 2 * i + slot
            for b in range(NBUF):
                _wait_gather(slot, b)
                _start_scatter(slot, b)

            @pl.when(g + 1 < NGRP)
            def _next_gathers():
                _idx_wait(g + 1, 1 - slot)
                for b in range(NBUF):
                    _wait_scatter(slot, b)
                    _start_gather(1 - slot, b)

            @pl.when(g + 2 < NGRP)
            def _prefetch():
                _idx_start(g + 2, slot)

        return carry

    lax.fori_loop(0, NGRP // 2, pair, 0)
    for b in range(NBUF):
        _wait_scatter(1, b)
    plsc.subcore_barrier()
    pltpu.sync_copy(agg_sp.at[pl.ds(s * ROWS_PT, ROWS_PT)],
                    out_hbm.at[c, pl.ds(s * ROWS_PT, ROWS_PT)])

    @pl.when(s == 0)
    def _write_tail():
        pltpu.sync_copy(agg_sp.at[pl.ds(NS * ROWS_PT, ROWS_TAIL)],
                        out_hbm.at[c, pl.ds(NS * ROWS_PT, ROWS_TAIL)])


_sc_aggregate = functools.partial(
    pl.kernel,
    out_type=jax.ShapeDtypeStruct((NC, N_NODES, D), jnp.float32),
    mesh=_MESH,
    scratch_types=[
        pltpu.VMEM((2, NBUF, CHUNK), jnp.int32),
        pltpu.VMEM((2, NBUF, CHUNK), jnp.int32),
        pltpu.VMEM((NBUF, CHUNK, D), jnp.float32),
        pltpu.VMEM_SHARED((N_NODES, D), jnp.float32),
        [pltpu.SemaphoreType.DMA] * 2,
        [pltpu.SemaphoreType.DMA] * 2,
        [pltpu.SemaphoreType.DMA] * NBUF,
        [pltpu.SemaphoreType.DMA] * NBUF,
    ],
)(_sc_body)


def _combine_body(act, p_ref, xr_ref, b_ref, o_ref):
    o_ref[...] = act(p_ref[0] + p_ref[1] + xr_ref[...] + b_ref[0][None, :])


def _tc_combine(p, xr, b, act, nb=5):
    blk = N_NODES // nb
    return pl.pallas_call(
        functools.partial(_combine_body, act),
        grid=(nb,),
        in_specs=[
            pl.BlockSpec((NC, blk, D), lambda i: (0, i, 0)),
            pl.BlockSpec((blk, D), lambda i: (i, 0)),
            pl.BlockSpec((1, D), lambda i: (0, 0)),
        ],
        out_specs=pl.BlockSpec((blk, D), lambda i: (i, 0)),
        out_shape=jax.ShapeDtypeStruct((N_NODES, D), jnp.float32),
    )(p, xr, b)


def _layer(x, wcat2d, b, gidx, dsti, zeros, act):
    hall, xr = _tc_matmul(x, wcat2d)
    p = _sc_aggregate(hall.reshape(R_CAT * N_NODES, D), gidx, dsti, zeros)
    return _tc_combine(p, xr, b.reshape(1, D), act)


def _flatten_w(W, root):
    wcat = jnp.concatenate([W, root[None]], axis=0)
    return wcat.transpose(1, 0, 2).reshape(D, R_CAT * D)


def kernel(x, edge_index, edge_type, W1, root1, b1, W2, root2, b2):
    src = edge_index[0].astype(jnp.int32)
    dst = edge_index[1].astype(jnp.int32)
    et = edge_type.astype(jnp.int32)
    gidx = (src * R_CAT + et).reshape(NW, NGRP, NBUF, CHUNK)
    dsti = dst.reshape(NW, NGRP, NBUF, CHUNK)
    zeros = jnp.zeros((N_NODES, D), jnp.float32)
    h = _layer(x, _flatten_w(W1, root1), b1, gidx, dsti, zeros,
               lambda v: jnp.maximum(v, 0.0))
    return _layer(h, _flatten_w(W2, root2), b2, gidx, dsti, zeros,
                  jax.nn.sigmoid)


# speedup vs baseline: 1.2823x; 1.2823x over previous
import functools

import jax
import jax.numpy as jnp
from jax import lax
from jax.experimental import pallas as pl
from jax.experimental.pallas import tpu as pltpu
from jax.experimental.pallas import tpu_sc as plsc

N_NODES = 10000
D = 128
N_REL = 8
E = 320000
R_CAT = N_REL + 1

NC, NS = 2, 16
NW = NC * NS
EPW = E // NW
CHUNK = 125
NCH = EPW // CHUNK
NROW = 2
NIDX = 4
UNROLL = 4
ROWS_PT = 624
ROWS_TAIL = N_NODES - NS * ROWS_PT

_MESH = plsc.VectorSubcoreMesh(core_axis_name="c", subcore_axis_name="s")


def _mm_body(x_ref, w_ref, o_ref):
    o_ref[0] = jnp.dot(x_ref[...], w_ref[0],
                       preferred_element_type=jnp.float32)


def _tc_matmul(x, wcat, nb=5):
    blk = N_NODES // nb
    return pl.pallas_call(
        _mm_body,
        grid=(R_CAT, nb),
        in_specs=[
            pl.BlockSpec((blk, D), lambda r, b: (b, 0)),
            pl.BlockSpec((1, D, D), lambda r, b: (r, 0, 0)),
        ],
        out_specs=pl.BlockSpec((1, blk, D), lambda r, b: (r, b, 0)),
        out_shape=jax.ShapeDtypeStruct((R_CAT, N_NODES, D), jnp.float32),
    )(x, wcat)


def _sc_body(hall, gidx_hbm, dst_hbm, zeros_hbm, out_hbm,
             ig_v, id_v, rows_v, agg_sp, igsems, idsems, gsems, ssems):
    c = lax.axis_index("c")
    s = lax.axis_index("s")
    wid = s * NC + c

    pltpu.sync_copy(zeros_hbm.at[pl.ds(s * ROWS_PT, ROWS_PT)],
                    agg_sp.at[pl.ds(s * ROWS_PT, ROWS_PT)])

    @pl.when(s == 0)
    def _init_tail():
        pltpu.sync_copy(zeros_hbm.at[pl.ds(NS * ROWS_PT, ROWS_TAIL)],
                        agg_sp.at[pl.ds(NS * ROWS_PT, ROWS_TAIL)])

    plsc.subcore_barrier()

    def _idx_start(j, q):
        pltpu.async_copy(gidx_hbm.at[wid, j], ig_v.at[q], igsems[q])
        pltpu.async_copy(dst_hbm.at[wid, j], id_v.at[q], idsems[q])

    def _idx_wait(j, q):
        pltpu.make_async_copy(gidx_hbm.at[wid, j], ig_v.at[q],
                              igsems[q]).wait()
        pltpu.make_async_copy(dst_hbm.at[wid, j], id_v.at[q],
                              idsems[q]).wait()

    def _start_gather(q, b):
        pltpu.async_copy(hall.at[ig_v.at[q, 0]], rows_v.at[b], gsems[b])

    def _wait_gather(q, b):
        pltpu.make_async_copy(hall.at[ig_v.at[q, 0]], rows_v.at[b],
                              gsems[b]).wait()

    def _start_scatter(q, b):
        pltpu.async_copy(rows_v.at[b], agg_sp.at[id_v.at[q, 0]],
                         ssems[b], add=True)

    def _wait_scatter(q, b):
        pltpu.make_async_copy(rows_v.at[b], agg_sp.at[id_v.at[q, 0]],
                              ssems[b]).wait()

    for q in range(NIDX - 1):
        _idx_start(q, q)
    _idx_wait(0, 0)
    _start_gather(0, 0)

    def step(i, carry):
        for k in range(UNROLL):
            j = i * UNROLL + k

            @pl.when(j >= 1)
            def _free_prev():
                _wait_scatter((k - 1) % NIDX, (k - 1) % NROW)

            @pl.when(j + 3 < NCH)
            def _prefetch():
                _idx_start(j + 3, (k + 3) % NIDX)

            @pl.when(j + 1 < NCH)
            def _lookahead():
                _idx_wait(j + 1, (k + 1) % NIDX)
                _start_gather((k + 1) % NIDX, (k + 1) % NROW)

            _wait_gather(k % NIDX, k % NROW)
            _start_scatter(k % NIDX, k % NROW)
        return carry

    lax.fori_loop(0, NCH // UNROLL, step, 0)
    _wait_scatter((NCH - 1) % NIDX, (NCH - 1) % NROW)
    plsc.subcore_barrier()
    pltpu.sync_copy(agg_sp.at[pl.ds(s * ROWS_PT, ROWS_PT)],
                    out_hbm.at[c, pl.ds(s * ROWS_PT, ROWS_PT)])

    @pl.when(s == 0)
    def _write_tail():
        pltpu.sync_copy(agg_sp.at[pl.ds(NS * ROWS_PT, ROWS_TAIL)],
                        out_hbm.at[c, pl.ds(NS * ROWS_PT, ROWS_TAIL)])


_sc_aggregate = functools.partial(
    pl.kernel,
    out_type=jax.ShapeDtypeStruct((NC, N_NODES, D), jnp.float32),
    mesh=_MESH,
    scratch_types=[
        pltpu.VMEM((NIDX, 1, CHUNK), jnp.int32),
        pltpu.VMEM((NIDX, 1, CHUNK), jnp.int32),
        pltpu.VMEM((NROW, CHUNK, D), jnp.float32),
        pltpu.VMEM_SHARED((N_NODES, D), jnp.float32),
        [pltpu.SemaphoreType.DMA] * NIDX,
        [pltpu.SemaphoreType.DMA] * NIDX,
        [pltpu.SemaphoreType.DMA] * NROW,
        [pltpu.SemaphoreType.DMA] * NROW,
    ],
)(_sc_body)


def _combine_body(act, p_ref, xr_ref, b_ref, o_ref):
    o_ref[...] = act(p_ref[0] + p_ref[1] + xr_ref[0] + b_ref[0][None, :])


def _tc_combine(p, hall, b, act, nb=5):
    blk = N_NODES // nb
    return pl.pallas_call(
        functools.partial(_combine_body, act),
        grid=(nb,),
        in_specs=[
            pl.BlockSpec((NC, blk, D), lambda i: (0, i, 0)),
            pl.BlockSpec((1, blk, D), lambda i: (R_CAT - 1, i, 0)),
            pl.BlockSpec((1, D), lambda i: (0, 0)),
        ],
        out_specs=pl.BlockSpec((blk, D), lambda i: (i, 0)),
        out_shape=jax.ShapeDtypeStruct((N_NODES, D), jnp.float32),
    )(p, hall, b)


def _layer(x, wcat, b, gidx, dsti, zeros, act):
    hall = _tc_matmul(x, wcat)
    p = _sc_aggregate(hall.reshape(R_CAT * N_NODES, D), gidx, dsti, zeros)
    return _tc_combine(p, hall, b.reshape(1, D), act)


def kernel(x, edge_index, edge_type, W1, root1, b1, W2, root2, b2):
    src = edge_index[0].astype(jnp.int32)
    dst = edge_index[1].astype(jnp.int32)
    et = edge_type.astype(jnp.int32)
    gidx = (et * N_NODES + src).reshape(NW, NCH, 1, CHUNK)
    dsti = dst.reshape(NW, NCH, 1, CHUNK)
    zeros = jnp.zeros((N_NODES, D), jnp.float32)
    wcat1 = jnp.concatenate([W1, root1[None]], axis=0)
    wcat2 = jnp.concatenate([W2, root2[None]], axis=0)
    h = _layer(x, wcat1, b1, gidx, dsti, zeros,
               lambda v: jnp.maximum(v, 0.0))
    return _layer(h, wcat2, b2, gidx, dsti, zeros, jax.nn.sigmoid)


# speedup vs baseline: 1.3362x; 1.0420x over previous
import functools

import jax
import jax.numpy as jnp
from jax import lax
from jax.experimental import pallas as pl
from jax.experimental.pallas import tpu as pltpu
from jax.experimental.pallas import tpu_sc as plsc

N_NODES = 10000
D = 128
N_REL = 8
E = 320000
R_CAT = N_REL + 1

NC, NS = 2, 16
NW = NC * NS
EPW = E // NW
CHUNK = 125
NCH = EPW // CHUNK
NROW = 2
NIDX = 4
UNROLL = 4
ROWS_PT = 624
ROWS_TAIL = N_NODES - NS * ROWS_PT

_MESH = plsc.VectorSubcoreMesh(core_axis_name="c", subcore_axis_name="s")


def _mm_body(x_ref, w_ref, o_ref):
    o_ref[0] = jnp.dot(x_ref[...], w_ref[0],
                       preferred_element_type=jnp.float32)


def _tc_matmul(x, wcat, nb=5):
    blk = N_NODES // nb
    return pl.pallas_call(
        _mm_body,
        grid=(R_CAT, nb),
        in_specs=[
            pl.BlockSpec((blk, D), lambda r, b: (b, 0)),
            pl.BlockSpec((1, D, D), lambda r, b: (r, 0, 0)),
        ],
        out_specs=pl.BlockSpec((1, blk, D), lambda r, b: (r, b, 0)),
        out_shape=jax.ShapeDtypeStruct((R_CAT, N_NODES, D), jnp.float32),
    )(x, wcat)


def _sc_body(hall, gidx_hbm, dst_hbm, zeros_hbm, out_hbm,
             ig_v, id_v, rows_v, agg_sp, igsems, idsems, gsems, ssems):
    c = lax.axis_index("c")
    s = lax.axis_index("s")
    wid = s * NC + c

    pltpu.sync_copy(zeros_hbm.at[pl.ds(s * ROWS_PT, ROWS_PT)],
                    agg_sp.at[pl.ds(s * ROWS_PT, ROWS_PT)])

    @pl.when(s == 0)
    def _init_tail():
        pltpu.sync_copy(zeros_hbm.at[pl.ds(NS * ROWS_PT, ROWS_TAIL)],
                        agg_sp.at[pl.ds(NS * ROWS_PT, ROWS_TAIL)])

    plsc.subcore_barrier()

    def _idx_start(j, q):
        pltpu.async_copy(gidx_hbm.at[wid, j], ig_v.at[q], igsems[q])
        pltpu.async_copy(dst_hbm.at[wid, j], id_v.at[q], idsems[q])

    def _idx_wait(j, q):
        pltpu.make_async_copy(gidx_hbm.at[wid, j], ig_v.at[q],
                              igsems[q]).wait()
        pltpu.make_async_copy(dst_hbm.at[wid, j], id_v.at[q],
                              idsems[q]).wait()

    def _start_gather(q, b):
        pltpu.async_copy(hall.at[ig_v.at[q, 0]], rows_v.at[b], gsems[b])

    def _wait_gather(q, b):
        pltpu.make_async_copy(hall.at[ig_v.at[q, 0]], rows_v.at[b],
                              gsems[b]).wait()

    def _start_scatter(q, b):
        pltpu.async_copy(rows_v.at[b], agg_sp.at[id_v.at[q, 0]],
                         ssems[b], add=True)

    def _wait_scatter(q, b):
        pltpu.make_async_copy(rows_v.at[b], agg_sp.at[id_v.at[q, 0]],
                              ssems[b]).wait()

    for q in range(NIDX - 1):
        _idx_start(q, q)
    _idx_wait(0, 0)
    _start_gather(0, 0)

    def step(i, carry):
        for k in range(UNROLL):
            j = i * UNROLL + k

            @pl.when(j >= 1)
            def _free_prev():
                _wait_scatter((k - 1) % NIDX, (k - 1) % NROW)

            @pl.when(j + 3 < NCH)
            def _prefetch():
                _idx_start(j + 3, (k + 3) % NIDX)

            @pl.when(j + 1 < NCH)
            def _lookahead():
                _idx_wait(j + 1, (k + 1) % NIDX)
                _start_gather((k + 1) % NIDX, (k + 1) % NROW)

            _wait_gather(k % NIDX, k % NROW)
            _start_scatter(k % NIDX, k % NROW)
        return carry

    lax.fori_loop(0, NCH // UNROLL, step, 0)
    _wait_scatter((NCH - 1) % NIDX, (NCH - 1) % NROW)
    plsc.subcore_barrier()
    pltpu.sync_copy(agg_sp.at[pl.ds(s * ROWS_PT, ROWS_PT)],
                    out_hbm.at[c, pl.ds(s * ROWS_PT, ROWS_PT)])

    @pl.when(s == 0)
    def _write_tail():
        pltpu.sync_copy(agg_sp.at[pl.ds(NS * ROWS_PT, ROWS_TAIL)],
                        out_hbm.at[c, pl.ds(NS * ROWS_PT, ROWS_TAIL)])


_sc_aggregate = functools.partial(
    pl.kernel,
    out_type=jax.ShapeDtypeStruct((NC, N_NODES, D), jnp.float32),
    mesh=_MESH,
    scratch_types=[
        pltpu.VMEM((NIDX, 1, CHUNK), jnp.int32),
        pltpu.VMEM((NIDX, 1, CHUNK), jnp.int32),
        pltpu.VMEM((NROW, CHUNK, D), jnp.float32),
        pltpu.VMEM_SHARED((N_NODES, D), jnp.float32),
        [pltpu.SemaphoreType.DMA] * NIDX,
        [pltpu.SemaphoreType.DMA] * NIDX,
        [pltpu.SemaphoreType.DMA] * NROW,
        [pltpu.SemaphoreType.DMA] * NROW,
    ],
)(_sc_body)


def _comb_mm_body(p_ref, xr_ref, b_ref, w_ref, o_ref, h_scr):
    @pl.when(pl.program_id(1) == 0)
    def _():
        h_scr[...] = jnp.maximum(
            p_ref[0] + p_ref[1] + xr_ref[0] + b_ref[0][None, :], 0.0)

    o_ref[0] = jnp.dot(h_scr[...], w_ref[0],
                       preferred_element_type=jnp.float32)


def _tc_combine_matmul(p, hall, b, wcat, nb=5):
    blk = N_NODES // nb
    return pl.pallas_call(
        _comb_mm_body,
        grid=(nb, R_CAT),
        in_specs=[
            pl.BlockSpec((NC, blk, D), lambda i, r: (0, i, 0)),
            pl.BlockSpec((1, blk, D), lambda i, r: (R_CAT - 1, i, 0)),
            pl.BlockSpec((1, D), lambda i, r: (0, 0)),
            pl.BlockSpec((1, D, D), lambda i, r: (r, 0, 0)),
        ],
        out_specs=pl.BlockSpec((1, blk, D), lambda i, r: (r, i, 0)),
        out_shape=jax.ShapeDtypeStruct((R_CAT, N_NODES, D), jnp.float32),
        scratch_shapes=[pltpu.VMEM((blk, D), jnp.float32)],
    )(p, hall, b, wcat)


def _combine_body(act, p_ref, xr_ref, b_ref, o_ref):
    o_ref[...] = act(p_ref[0] + p_ref[1] + xr_ref[0] + b_ref[0][None, :])


def _tc_combine(p, hall, b, act, nb=5):
    blk = N_NODES // nb
    return pl.pallas_call(
        functools.partial(_combine_body, act),
        grid=(nb,),
        in_specs=[
            pl.BlockSpec((NC, blk, D), lambda i: (0, i, 0)),
            pl.BlockSpec((1, blk, D), lambda i: (R_CAT - 1, i, 0)),
            pl.BlockSpec((1, D), lambda i: (0, 0)),
        ],
        out_specs=pl.BlockSpec((blk, D), lambda i: (i, 0)),
        out_shape=jax.ShapeDtypeStruct((N_NODES, D), jnp.float32),
    )(p, hall, b)


def kernel(x, edge_index, edge_type, W1, root1, b1, W2, root2, b2):
    src = edge_index[0].astype(jnp.int32)
    dst = edge_index[1].astype(jnp.int32)
    et = edge_type.astype(jnp.int32)
    gidx = (et * N_NODES + src).reshape(NW, NCH, 1, CHUNK)
    dsti = dst.reshape(NW, NCH, 1, CHUNK)
    zeros = jnp.zeros((N_NODES, D), jnp.float32)
    wcat1 = jnp.concatenate([W1, root1[None]], axis=0)
    wcat2 = jnp.concatenate([W2, root2[None]], axis=0)
    hall1 = _tc_matmul(x, wcat1)
    p1 = _sc_aggregate(hall1.reshape(R_CAT * N_NODES, D), gidx, dsti, zeros)
    hall2 = _tc_combine_matmul(p1, hall1, b1.reshape(1, D), wcat2)
    p2 = _sc_aggregate(hall2.reshape(R_CAT * N_NODES, D), gidx, dsti, zeros)
    return _tc_combine(p2, hall2, b2.reshape(1, D), jax.nn.sigmoid)


# speedup vs baseline: 1.5817x; 1.1838x over previous
import functools

import jax
import jax.numpy as jnp
from jax import lax
from jax.experimental import pallas as pl
from jax.experimental.pallas import tpu as pltpu
from jax.experimental.pallas import tpu_sc as plsc

N_NODES = 10000
D = 128
N_REL = 8
E = 320000
R_CAT = N_REL + 1

NC, NS = 2, 16
NW = NC * NS
EPW = E // NW
CHUNK = 125
NCH = EPW // CHUNK
NROW = 2
NIDX = 4
UNROLL = 4
ROWS_PT = 624
ROWS_TAIL = N_NODES - NS * ROWS_PT

_MESH = plsc.VectorSubcoreMesh(core_axis_name="c", subcore_axis_name="s")


def _mm_body(x_ref, w_ref, o_ref):
    o_ref[0] = jnp.dot(x_ref[...], w_ref[0],
                       preferred_element_type=jnp.float32)


def _tc_matmul(x, wcat, nb=1):
    blk = N_NODES // nb
    return pl.pallas_call(
        _mm_body,
        grid=(R_CAT, nb),
        in_specs=[
            pl.BlockSpec((blk, D), lambda r, b: (b, 0)),
            pl.BlockSpec((1, D, D), lambda r, b: (r, 0, 0)),
        ],
        out_specs=pl.BlockSpec((1, blk, D), lambda r, b: (r, b, 0)),
        out_shape=jax.ShapeDtypeStruct((R_CAT, N_NODES, D), jnp.float32),
    )(x, wcat)


def _sc_body(hall, gidx_hbm, dst_hbm, zeros_hbm, out_hbm,
             ig_v, id_v, rows_v, agg_sp, igsems, idsems, gsems, ssems):
    c = lax.axis_index("c")
    s = lax.axis_index("s")
    wid = s * NC + c

    pltpu.sync_copy(zeros_hbm.at[pl.ds(s * ROWS_PT, ROWS_PT)],
                    agg_sp.at[pl.ds(s * ROWS_PT, ROWS_PT)])

    @pl.when(s == 0)
    def _init_tail():
        pltpu.sync_copy(zeros_hbm.at[pl.ds(NS * ROWS_PT, ROWS_TAIL)],
                        agg_sp.at[pl.ds(NS * ROWS_PT, ROWS_TAIL)])

    plsc.subcore_barrier()

    def _idx_start(j, q):
        pltpu.async_copy(gidx_hbm.at[wid, j], ig_v.at[q], igsems[q])
        pltpu.async_copy(dst_hbm.at[wid, j], id_v.at[q], idsems[q])

    def _idx_wait(j, q):
        pltpu.make_async_copy(gidx_hbm.at[wid, j], ig_v.at[q],
                              igsems[q]).wait()
        pltpu.make_async_copy(dst_hbm.at[wid, j], id_v.at[q],
                              idsems[q]).wait()

    def _start_gather(q, b):
        pltpu.async_copy(hall.at[ig_v.at[q, 0]], rows_v.at[b], gsems[b])

    def _wait_gather(q, b):
        pltpu.make_async_copy(hall.at[ig_v.at[q, 0]], rows_v.at[b],
                              gsems[b]).wait()

    def _start_scatter(q, b):
        pltpu.async_copy(rows_v.at[b], agg_sp.at[id_v.at[q, 0]],
                         ssems[b], add=True)

    def _wait_scatter(q, b):
        pltpu.make_async_copy(rows_v.at[b], agg_sp.at[id_v.at[q, 0]],
                              ssems[b]).wait()

    for q in range(NIDX - 1):
        _idx_start(q, q)
    _idx_wait(0, 0)
    _start_gather(0, 0)

    def step(i, carry):
        for k in range(UNROLL):
            j = i * UNROLL + k

            @pl.when(j >= 1)
            def _free_prev():
                _wait_scatter((k - 1) % NIDX, (k - 1) % NROW)

            @pl.when(j + 3 < NCH)
            def _prefetch():
                _idx_start(j + 3, (k + 3) % NIDX)

            @pl.when(j + 1 < NCH)
            def _lookahead():
                _idx_wait(j + 1, (k + 1) % NIDX)
                _start_gather((k + 1) % NIDX, (k + 1) % NROW)

            _wait_gather(k % NIDX, k % NROW)
            _start_scatter(k % NIDX, k % NROW)
        return carry

    lax.fori_loop(0, NCH // UNROLL, step, 0)
    _wait_scatter((NCH - 1) % NIDX, (NCH - 1) % NROW)
    plsc.subcore_barrier()
    pltpu.sync_copy(agg_sp.at[pl.ds(s * ROWS_PT, ROWS_PT)],
                    out_hbm.at[c, pl.ds(s * ROWS_PT, ROWS_PT)])

    @pl.when(s == 0)
    def _write_tail():
        pltpu.sync_copy(agg_sp.at[pl.ds(NS * ROWS_PT, ROWS_TAIL)],
                        out_hbm.at[c, pl.ds(NS * ROWS_PT, ROWS_TAIL)])


_sc_aggregate = functools.partial(
    pl.kernel,
    out_type=jax.ShapeDtypeStruct((NC, N_NODES, D), jnp.float32),
    mesh=_MESH,
    scratch_types=[
        pltpu.VMEM((NIDX, 1, CHUNK), jnp.int32),
        pltpu.VMEM((NIDX, 1, CHUNK), jnp.int32),
        pltpu.VMEM((NROW, CHUNK, D), jnp.float32),
        pltpu.VMEM_SHARED((N_NODES, D), jnp.float32),
        [pltpu.SemaphoreType.DMA] * NIDX,
        [pltpu.SemaphoreType.DMA] * NIDX,
        [pltpu.SemaphoreType.DMA] * NROW,
        [pltpu.SemaphoreType.DMA] * NROW,
    ],
)(_sc_body)


def _comb_mm_body(p_ref, xr_ref, b_ref, w_ref, o_ref, h_scr):
    @pl.when(pl.program_id(1) == 0)
    def _():
        h_scr[...] = jnp.maximum(
            p_ref[0] + p_ref[1] + xr_ref[0] + b_ref[0][None, :], 0.0)

    o_ref[0] = jnp.dot(h_scr[...], w_ref[0],
                       preferred_element_type=jnp.float32)


def _tc_combine_matmul(p, hall, b, wcat, nb=1):
    blk = N_NODES // nb
    return pl.pallas_call(
        _comb_mm_body,
        grid=(nb, R_CAT),
        in_specs=[
            pl.BlockSpec((NC, blk, D), lambda i, r: (0, i, 0)),
            pl.BlockSpec((1, blk, D), lambda i, r: (R_CAT - 1, i, 0)),
            pl.BlockSpec((1, D), lambda i, r: (0, 0)),
            pl.BlockSpec((1, D, D), lambda i, r: (r, 0, 0)),
        ],
        out_specs=pl.BlockSpec((1, blk, D), lambda i, r: (r, i, 0)),
        out_shape=jax.ShapeDtypeStruct((R_CAT, N_NODES, D), jnp.float32),
        scratch_shapes=[pltpu.VMEM((blk, D), jnp.float32)],
    )(p, hall, b, wcat)


def _combine_body(act, p_ref, xr_ref, b_ref, o_ref):
    o_ref[...] = act(p_ref[0] + p_ref[1] + xr_ref[0] + b_ref[0][None, :])


def _tc_combine(p, hall, b, act, nb=1):
    blk = N_NODES // nb
    return pl.pallas_call(
        functools.partial(_combine_body, act),
        grid=(nb,),
        in_specs=[
            pl.BlockSpec((NC, blk, D), lambda i: (0, i, 0)),
            pl.BlockSpec((1, blk, D), lambda i: (R_CAT - 1, i, 0)),
            pl.BlockSpec((1, D), lambda i: (0, 0)),
        ],
        out_specs=pl.BlockSpec((blk, D), lambda i: (i, 0)),
        out_shape=jax.ShapeDtypeStruct((N_NODES, D), jnp.float32),
    )(p, hall, b)


def kernel(x, edge_index, edge_type, W1, root1, b1, W2, root2, b2):
    src = edge_index[0].astype(jnp.int32)
    dst = edge_index[1].astype(jnp.int32)
    et = edge_type.astype(jnp.int32)
    gidx = (et * N_NODES + src).reshape(NW, NCH, 1, CHUNK)
    dsti = dst.reshape(NW, NCH, 1, CHUNK)
    zeros = jnp.zeros((N_NODES, D), jnp.float32)
    wcat1 = jnp.concatenate([W1, root1[None]], axis=0)
    wcat2 = jnp.concatenate([W2, root2[None]], axis=0)
    hall1 = _tc_matmul(x, wcat1)
    p1 = _sc_aggregate(hall1.reshape(R_CAT * N_NODES, D), gidx, dsti, zeros)
    hall2 = _tc_combine_matmul(p1, hall1, b1.reshape(1, D), wcat2)
    p2 = _sc_aggregate(hall2.reshape(R_CAT * N_NODES, D), gidx, dsti, zeros)
    return _tc_combine(p2, hall2, b2.reshape(1, D), jax.nn.sigmoid)


# speedup vs baseline: 1.6026x; 1.0132x over previous
import functools

import jax
import jax.numpy as jnp
from jax import lax
from jax.experimental import pallas as pl
from jax.experimental.pallas import tpu as pltpu
from jax.experimental.pallas import tpu_sc as plsc

N_NODES = 10000
D = 128
N_REL = 8
E = 320000
R_CAT = N_REL + 1

NC, NS = 2, 16
NW = NC * NS
EPW = E // NW
CHUNK = 125
NCH = EPW // CHUNK
NROW = 2
NIDX = 4
UNROLL = 4
ROWS_PT = 624
ROWS_TAIL = N_NODES - NS * ROWS_PT

_MESH = plsc.VectorSubcoreMesh(core_axis_name="c", subcore_axis_name="s")


def _mm_body(x_ref, w_ref, root_ref, o_ref):
    r = pl.program_id(0)

    @pl.when(r < R_CAT - 1)
    def _rel():
        o_ref[0] = jnp.dot(x_ref[...], w_ref[0],
                           preferred_element_type=jnp.float32)

    @pl.when(r == R_CAT - 1)
    def _root():
        o_ref[0] = jnp.dot(x_ref[...], root_ref[...],
                           preferred_element_type=jnp.float32)


def _tc_matmul(x, w, root):
    return pl.pallas_call(
        _mm_body,
        grid=(R_CAT,),
        in_specs=[
            pl.BlockSpec((N_NODES, D), lambda r: (0, 0)),
            pl.BlockSpec((1, D, D), lambda r: (jnp.minimum(r, N_REL - 1), 0, 0)),
            pl.BlockSpec((D, D), lambda r: (0, 0)),
        ],
        out_specs=pl.BlockSpec((1, N_NODES, D), lambda r: (r, 0, 0)),
        out_shape=jax.ShapeDtypeStruct((R_CAT, N_NODES, D), jnp.float32),
    )(x, w, root)


def _sc_body(hall, gidx_hbm, dst_hbm, zeros_hbm, out_hbm,
             ig_v, id_v, rows_v, agg_sp, igsems, idsems, gsems, ssems):
    c = lax.axis_index("c")
    s = lax.axis_index("s")
    wid = s * NC + c

    xr_base = (R_CAT - 1) * N_NODES

    @pl.when(c == 0)
    def _init_xr():
        pltpu.sync_copy(hall.at[pl.ds(xr_base + s * ROWS_PT, ROWS_PT)],
                        agg_sp.at[pl.ds(s * ROWS_PT, ROWS_PT)])

        @pl.when(s == 0)
        def _tail():
            pltpu.sync_copy(hall.at[pl.ds(xr_base + NS * ROWS_PT, ROWS_TAIL)],
                            agg_sp.at[pl.ds(NS * ROWS_PT, ROWS_TAIL)])

    @pl.when(c != 0)
    def _init_zero():
        pltpu.sync_copy(zeros_hbm.at[pl.ds(s * ROWS_PT, ROWS_PT)],
                        agg_sp.at[pl.ds(s * ROWS_PT, ROWS_PT)])

        @pl.when(s == 0)
        def _tail():
            pltpu.sync_copy(zeros_hbm.at[pl.ds(NS * ROWS_PT, ROWS_TAIL)],
                            agg_sp.at[pl.ds(NS * ROWS_PT, ROWS_TAIL)])

    plsc.subcore_barrier()

    def _idx_start(j, q):
        pltpu.async_copy(gidx_hbm.at[wid, j], ig_v.at[q], igsems[q])
        pltpu.async_copy(dst_hbm.at[wid, j], id_v.at[q], idsems[q])

    def _idx_wait(j, q):
        pltpu.make_async_copy(gidx_hbm.at[wid, j], ig_v.at[q],
                              igsems[q]).wait()
        pltpu.make_async_copy(dst_hbm.at[wid, j], id_v.at[q],
                              idsems[q]).wait()

    def _start_gather(q, b):
        pltpu.async_copy(hall.at[ig_v.at[q, 0]], rows_v.at[b], gsems[b])

    def _wait_gather(q, b):
        pltpu.make_async_copy(hall.at[ig_v.at[q, 0]], rows_v.at[b],
                              gsems[b]).wait()

    def _start_scatter(q, b):
        pltpu.async_copy(rows_v.at[b], agg_sp.at[id_v.at[q, 0]],
                         ssems[b], add=True)

    def _wait_scatter(q, b):
        pltpu.make_async_copy(rows_v.at[b], agg_sp.at[id_v.at[q, 0]],
                              ssems[b]).wait()

    for q in range(NIDX - 1):
        _idx_start(q, q)
    _idx_wait(0, 0)
    _start_gather(0, 0)

    def step(i, carry):
        for k in range(UNROLL):
            j = i * UNROLL + k

            @pl.when(j >= 1)
            def _free_prev():
                _wait_scatter((k - 1) % NIDX, (k - 1) % NROW)

            @pl.when(j + 3 < NCH)
            def _prefetch():
                _idx_start(j + 3, (k + 3) % NIDX)

            @pl.when(j + 1 < NCH)
            def _lookahead():
                _idx_wait(j + 1, (k + 1) % NIDX)
                _start_gather((k + 1) % NIDX, (k + 1) % NROW)

            _wait_gather(k % NIDX, k % NROW)
            _start_scatter(k % NIDX, k % NROW)
        return carry

    lax.fori_loop(0, NCH // UNROLL, step, 0)
    _wait_scatter((NCH - 1) % NIDX, (NCH - 1) % NROW)
    plsc.subcore_barrier()
    pltpu.sync_copy(agg_sp.at[pl.ds(s * ROWS_PT, ROWS_PT)],
                    out_hbm.at[c, pl.ds(s * ROWS_PT, ROWS_PT)])

    @pl.when(s == 0)
    def _write_tail():
        pltpu.sync_copy(agg_sp.at[pl.ds(NS * ROWS_PT, ROWS_TAIL)],
                        out_hbm.at[c, pl.ds(NS * ROWS_PT, ROWS_TAIL)])


_sc_aggregate = functools.partial(
    pl.kernel,
    out_type=jax.ShapeDtypeStruct((NC, N_NODES, D), jnp.float32),
    mesh=_MESH,
    scratch_types=[
        pltpu.VMEM((NIDX, 1, CHUNK), jnp.int32),
        pltpu.VMEM((NIDX, 1, CHUNK), jnp.int32),
        pltpu.VMEM((NROW, CHUNK, D), jnp.float32),
        pltpu.VMEM_SHARED((N_NODES, D), jnp.float32),
        [pltpu.SemaphoreType.DMA] * NIDX,
        [pltpu.SemaphoreType.DMA] * NIDX,
        [pltpu.SemaphoreType.DMA] * NROW,
        [pltpu.SemaphoreType.DMA] * NROW,
    ],
)(_sc_body)


def _comb_mm_body(p_ref, b_ref, w_ref, root_ref, o_ref, h_scr):
    r = pl.program_id(0)

    @pl.when(r == 0)
    def _():
        h_scr[...] = jnp.maximum(
            p_ref[0] + p_ref[1] + b_ref[0][None, :], 0.0)

    @pl.when(r < R_CAT - 1)
    def _rel():
        o_ref[0] = jnp.dot(h_scr[...], w_ref[0],
                           preferred_element_type=jnp.float32)

    @pl.when(r == R_CAT - 1)
    def _root():
        o_ref[0] = jnp.dot(h_scr[...], root_ref[...],
                           preferred_element_type=jnp.float32)


def _tc_combine_matmul(p, b, w, root):
    return pl.pallas_call(
        _comb_mm_body,
        grid=(R_CAT,),
        in_specs=[
            pl.BlockSpec((NC, N_NODES, D), lambda r: (0, 0, 0)),
            pl.BlockSpec((1, D), lambda r: (0, 0)),
            pl.BlockSpec((1, D, D), lambda r: (jnp.minimum(r, N_REL - 1), 0, 0)),
            pl.BlockSpec((D, D), lambda r: (0, 0)),
        ],
        out_specs=pl.BlockSpec((1, N_NODES, D), lambda r: (r, 0, 0)),
        out_shape=jax.ShapeDtypeStruct((R_CAT, N_NODES, D), jnp.float32),
        scratch_shapes=[pltpu.VMEM((N_NODES, D), jnp.float32)],
    )(p, b, w, root)


def _combine_body(p_ref, b_ref, o_ref):
    o_ref[...] = jax.nn.sigmoid(p_ref[0] + p_ref[1] + b_ref[0][None, :])


def _tc_combine(p, b):
    return pl.pallas_call(
        _combine_body,
        grid=(1,),
        in_specs=[
            pl.BlockSpec((NC, N_NODES, D), lambda i: (0, 0, 0)),
            pl.BlockSpec((1, D), lambda i: (0, 0)),
        ],
        out_specs=pl.BlockSpec((N_NODES, D), lambda i: (0, 0)),
        out_shape=jax.ShapeDtypeStruct((N_NODES, D), jnp.float32),
    )(p, b)


def kernel(x, edge_index, edge_type, W1, root1, b1, W2, root2, b2):
    src = edge_index[0].astype(jnp.int32)
    dst = edge_index[1].astype(jnp.int32)
    et = edge_type.astype(jnp.int32)
    gidx = (et * N_NODES + src).reshape(NW, NCH, 1, CHUNK)
    dsti = dst.reshape(NW, NCH, 1, CHUNK)
    zeros = jnp.zeros((N_NODES, D), jnp.float32)
    hall1 = _tc_matmul(x, W1, root1)
    p1 = _sc_aggregate(hall1.reshape(R_CAT * N_NODES, D), gidx, dsti, zeros)
    hall2 = _tc_combine_matmul(p1, b1.reshape(1, D), W2, root2)
    p2 = _sc_aggregate(hall2.reshape(R_CAT * N_NODES, D), gidx, dsti, zeros)
    return _tc_combine(p2, b2.reshape(1, D))
